# baseline (speedup 1.0000x reference)
import functools

import jax
import jax.numpy as jnp
from jax import lax
from jax.experimental import pallas as pl
from jax.experimental.pallas import tpu as pltpu
from jax.experimental.pallas import tpu_sc as plsc

D = 768
L = 16
NC = 2
NS = 16
NW = NC * NS
CHUNK = 16
RING = 4
PF = 2


def _make_sc_kernel(N):
    rows_per_w = N // NW
    steps = rows_per_w // CHUNK
    g_iters = steps // RING
    mesh = plsc.VectorSubcoreMesh(core_axis_name="c", subcore_axis_name="s")

    scratch = (
        [pltpu.VMEM((rows_per_w,), jnp.int32)]
        + [pltpu.VMEM((CHUNK, D), jnp.float32)] * (2 * RING)
        + [pltpu.SemaphoreType.DMA] * (4 * RING)
    )

    @functools.partial(
        pl.kernel,
        mesh=mesh,
        out_type=(
            jax.ShapeDtypeStruct((N, D), jnp.float32),
            jax.ShapeDtypeStruct((N, D), jnp.float32),
        ),
        scratch_types=scratch,
    )
    def sc_kernel(x_hbm, idx_hbm, tab_hbm, comp_hbm, res_hbm, *scr):
        idx_all = scr[0]
        x_bufs = scr[1:1 + RING]
        r_bufs = scr[1 + RING:1 + 2 * RING]
        sems = scr[1 + 2 * RING:]
        in_x = sems[0:RING]
        in_g = sems[RING:2 * RING]
        out_c = sems[2 * RING:3 * RING]
        out_r = sems[3 * RING:]

        wid = lax.axis_index("s") * NC + lax.axis_index("c")
        base = wid * rows_per_w
        pltpu.sync_copy(idx_hbm.at[pl.ds(base, rows_per_w)], idx_all)

        def issue_inputs(g, b):
            r0 = base + g * CHUNK
            pltpu.async_copy(
                tab_hbm.at[idx_all.at[pl.ds(g * CHUNK, CHUNK)]],
                r_bufs[b], in_g[b])
            pltpu.async_copy(x_hbm.at[pl.ds(r0, CHUNK)], x_bufs[b], in_x[b])

        def wait_inputs(b):
            pltpu.make_async_copy(
                x_hbm.at[pl.ds(0, CHUNK)], x_bufs[b], in_x[b]).wait()
            pltpu.make_async_copy(
                tab_hbm.at[idx_all.at[pl.ds(0, CHUNK)]],
                r_bufs[b], in_g[b]).wait()

        def issue_outputs(g, b):
            r0 = base + g * CHUNK
            pltpu.async_copy(r_bufs[b], comp_hbm.at[pl.ds(r0, CHUNK)], out_c[b])
            pltpu.async_copy(x_bufs[b], res_hbm.at[pl.ds(r0, CHUNK)], out_r[b])

        def wait_outputs(b):
            pltpu.make_async_copy(
                r_bufs[b], comp_hbm.at[pl.ds(0, CHUNK)], out_c[b]).wait()
            pltpu.make_async_copy(
                x_bufs[b], res_hbm.at[pl.ds(0, CHUNK)], out_r[b]).wait()

        def compute(b):
            def row(r, c):
                def col(j, c2):
                    sl = pl.ds(j * L, L)
                    x_bufs[b][r, sl] = x_bufs[b][r, sl] - r_bufs[b][r, sl]
                    return c2

                return lax.fori_loop(0, D // L, col, c, unroll=8)

            lax.fori_loop(0, CHUNK, row, 0)

        for p in range(PF):
            issue_inputs(p, p)

        def giter(G, carry):
            for b in range(RING):
                g = G * RING + b
                gp = g + PF
                pb = (b + PF) % RING
                wait_inputs(b)
                @pl.when(gp < steps)
                def _():
                    @pl.when(gp >= RING)
                    def _():
                        wait_outputs(pb)

                    issue_inputs(gp, pb)

                compute(b)
                issue_outputs(g, b)
            return carry

        lax.fori_loop(0, g_iters, giter, 0)
        for b in range(RING):
            wait_outputs(b)

    return sc_kernel


def kernel(x, cycle_indices, learnable_cycles):
    B, S, d = x.shape
    N = B * S
    x2 = x.reshape(N, d)
    idx = cycle_indices.reshape(N).astype(jnp.int32)
    comp, res = _make_sc_kernel(N)(x2, idx, learnable_cycles)
    return comp.reshape(B, S, d), res.reshape(B, S, d)


# speedup vs baseline: 1.0154x; 1.0154x over previous
import functools

import jax
import jax.numpy as jnp
from jax import lax
from jax.experimental import pallas as pl
from jax.experimental.pallas import tpu as pltpu
from jax.experimental.pallas import tpu_sc as plsc

D = 768
L = 16
NC = 2
NS = 16
NW = NC * NS
CHUNK = 8
RING = 4
PF = 2


def _make_sc_kernel(N):
    rows_per_w = N // NW
    steps = rows_per_w // CHUNK
    g_iters = steps // RING
    mesh = plsc.VectorSubcoreMesh(core_axis_name="c", subcore_axis_name="s")

    scratch = (
        [pltpu.VMEM((rows_per_w,), jnp.int32)]
        + [pltpu.VMEM((CHUNK, D), jnp.float32)] * (2 * RING)
        + [pltpu.SemaphoreType.DMA] * (4 * RING)
    )

    @functools.partial(
        pl.kernel,
        mesh=mesh,
        out_type=(
            jax.ShapeDtypeStruct((N, D), jnp.float32),
            jax.ShapeDtypeStruct((N, D), jnp.float32),
        ),
        scratch_types=scratch,
    )
    def sc_kernel(x_hbm, idx_hbm, tab_hbm, comp_hbm, res_hbm, *scr):
        idx_all = scr[0]
        x_bufs = scr[1:1 + RING]
        r_bufs = scr[1 + RING:1 + 2 * RING]
        sems = scr[1 + 2 * RING:]
        in_x = sems[0:RING]
        in_g = sems[RING:2 * RING]
        out_c = sems[2 * RING:3 * RING]
        out_r = sems[3 * RING:]

        wid = lax.axis_index("s") * NC + lax.axis_index("c")
        base = wid * rows_per_w
        pltpu.sync_copy(idx_hbm.at[pl.ds(base, rows_per_w)], idx_all)

        def issue_inputs(g, b):
            r0 = base + g * CHUNK
            pltpu.async_copy(
                tab_hbm.at[idx_all.at[pl.ds(g * CHUNK, CHUNK)]],
                r_bufs[b], in_g[b])
            pltpu.async_copy(x_hbm.at[pl.ds(r0, CHUNK)], x_bufs[b], in_x[b])

        def wait_inputs(b):
            pltpu.make_async_copy(
                x_hbm.at[pl.ds(0, CHUNK)], x_bufs[b], in_x[b]).wait()
            pltpu.make_async_copy(
                tab_hbm.at[idx_all.at[pl.ds(0, CHUNK)]],
                r_bufs[b], in_g[b]).wait()

        def issue_outputs(g, b):
            r0 = base + g * CHUNK
            pltpu.async_copy(r_bufs[b], comp_hbm.at[pl.ds(r0, CHUNK)], out_c[b])
            pltpu.async_copy(x_bufs[b], res_hbm.at[pl.ds(r0, CHUNK)], out_r[b])

        def wait_outputs(b):
            pltpu.make_async_copy(
                r_bufs[b], comp_hbm.at[pl.ds(0, CHUNK)], out_c[b]).wait()
            pltpu.make_async_copy(
                x_bufs[b], res_hbm.at[pl.ds(0, CHUNK)], out_r[b]).wait()

        def compute(b):
            def row(r, c):
                def col(j, c2):
                    sl = pl.ds(j * L, L)
                    x_bufs[b][r, sl] = x_bufs[b][r, sl] - r_bufs[b][r, sl]
                    return c2

                return lax.fori_loop(0, D // L, col, c, unroll=8)

            lax.fori_loop(0, CHUNK, row, 0)

        for p in range(PF):
            issue_inputs(p, p)

        def giter(G, carry):
            for b in range(RING):
                g = G * RING + b
                gp = g + PF
                pb = (b + PF) % RING
                wait_inputs(b)
                @pl.when(gp < steps)
                def _():
                    @pl.when(gp >= RING)
                    def _():
                        wait_outputs(pb)

                    issue_inputs(gp, pb)

                compute(b)
                issue_outputs(g, b)
            return carry

        lax.fori_loop(0, g_iters, giter, 0)
        for b in range(RING):
            wait_outputs(b)

    return sc_kernel


def kernel(x, cycle_indices, learnable_cycles):
    B, S, d = x.shape
    N = B * S
    x2 = x.reshape(N, d)
    idx = cycle_indices.reshape(N).astype(jnp.int32)
    comp, res = _make_sc_kernel(N)(x2, idx, learnable_cycles)
    return comp.reshape(B, S, d), res.reshape(B, S, d)


# speedup vs baseline: 1.0222x; 1.0067x over previous
import functools

import jax
import jax.numpy as jnp
from jax import lax
from jax.experimental import pallas as pl
from jax.experimental.pallas import tpu as pltpu
from jax.experimental.pallas import tpu_sc as plsc

D = 768
L = 16
NC = 2
NS = 16
NW = NC * NS
CHUNK = 8
RING = 4
PF = 2


def _make_sc_kernel(N):
    rows_per_w = N // NW
    steps = rows_per_w // CHUNK
    g_iters = steps // RING
    mesh = plsc.VectorSubcoreMesh(core_axis_name="c", subcore_axis_name="s")

    scratch = (
        [pltpu.VMEM((rows_per_w,), jnp.int32)]
        + [pltpu.VMEM((CHUNK, D), jnp.float32)] * (2 * RING)
        + [pltpu.SemaphoreType.DMA] * (4 * RING)
    )

    @functools.partial(
        pl.kernel,
        mesh=mesh,
        out_type=(
            jax.ShapeDtypeStruct((N, D), jnp.float32),
            jax.ShapeDtypeStruct((N, D), jnp.float32),
        ),
        scratch_types=scratch,
    )
    def sc_kernel(x_hbm, idx_hbm, tab_hbm, comp_hbm, res_hbm, *scr):
        idx_all = scr[0]
        x_bufs = scr[1:1 + RING]
        r_bufs = scr[1 + RING:1 + 2 * RING]
        sems = scr[1 + 2 * RING:]
        in_x = sems[0:RING]
        in_g = sems[RING:2 * RING]
        out_c = sems[2 * RING:3 * RING]
        out_r = sems[3 * RING:]

        wid = lax.axis_index("s") * NC + lax.axis_index("c")
        base = wid * rows_per_w
        pltpu.sync_copy(idx_hbm.at[pl.ds(base, rows_per_w)], idx_all)

        def issue_inputs(g, b):
            r0 = base + g * CHUNK
            pltpu.async_copy(
                tab_hbm.at[idx_all.at[pl.ds(g * CHUNK, CHUNK)]],
                r_bufs[b], in_g[b])
            pltpu.async_copy(x_hbm.at[pl.ds(r0, CHUNK)], x_bufs[b], in_x[b])

        def wait_inputs(b):
            pltpu.make_async_copy(
                x_hbm.at[pl.ds(0, CHUNK)], x_bufs[b], in_x[b]).wait()
            pltpu.make_async_copy(
                tab_hbm.at[idx_all.at[pl.ds(0, CHUNK)]],
                r_bufs[b], in_g[b]).wait()

        def issue_outputs(g, b):
            r0 = base + g * CHUNK
            pltpu.async_copy(r_bufs[b], comp_hbm.at[pl.ds(r0, CHUNK)], out_c[b])
            pltpu.async_copy(x_bufs[b], res_hbm.at[pl.ds(r0, CHUNK)], out_r[b])

        def wait_outputs(b):
            pltpu.make_async_copy(
                r_bufs[b], comp_hbm.at[pl.ds(0, CHUNK)], out_c[b]).wait()
            pltpu.make_async_copy(
                x_bufs[b], res_hbm.at[pl.ds(0, CHUNK)], out_r[b]).wait()

        def compute(b):
            def row(r, c):
                def col(j, c2):
                    sl = pl.ds(j * L, L)
                    x_bufs[b][r, sl] = x_bufs[b][r, sl] - r_bufs[b][r, sl]
                    return c2

                return lax.fori_loop(0, D // L, col, c, unroll=16)

            lax.fori_loop(0, CHUNK, row, 0)

        for p in range(PF):
            issue_inputs(p, p)

        def giter(G, carry):
            for b in range(RING):
                g = G * RING + b
                gp = g + PF
                pb = (b + PF) % RING
                wait_inputs(b)
                @pl.when(gp < steps)
                def _():
                    @pl.when(gp >= RING)
                    def _():
                        wait_outputs(pb)

                    issue_inputs(gp, pb)

                compute(b)
                issue_outputs(g, b)
            return carry

        lax.fori_loop(0, g_iters, giter, 0)
        for b in range(RING):
            wait_outputs(b)

    return sc_kernel


def kernel(x, cycle_indices, learnable_cycles):
    B, S, d = x.shape
    N = B * S
    x2 = x.reshape(N, d)
    idx = cycle_indices.reshape(N).astype(jnp.int32)
    comp, res = _make_sc_kernel(N)(x2, idx, learnable_cycles)
    return comp.reshape(B, S, d), res.reshape(B, S, d)


# speedup vs baseline: 1.4925x; 1.4601x over previous
import functools

import jax
import jax.numpy as jnp
from jax import lax
from jax.experimental import pallas as pl
from jax.experimental.pallas import tpu as pltpu
from jax.experimental.pallas import tpu_sc as plsc

D = 768
L = 16
NC = 2
NS = 16
NW = NC * NS
CHUNK = 8
RING = 4
PF = 2


def _make_sc_kernel(N):
    rows_per_w = N // NW
    steps = rows_per_w // CHUNK
    g_iters = steps // RING
    mesh = plsc.VectorSubcoreMesh(core_axis_name="c", subcore_axis_name="s")

    scratch = (
        [pltpu.VMEM((rows_per_w,), jnp.int32)]
        + [pltpu.VMEM((CHUNK, D), jnp.float32)] * (2 * RING)
        + [pltpu.SemaphoreType.DMA] * (4 * RING)
    )

    @functools.partial(
        pl.kernel,
        mesh=mesh,
        out_type=(
            jax.ShapeDtypeStruct((N, D), jnp.float32),
            jax.ShapeDtypeStruct((N, D), jnp.float32),
        ),
        scratch_types=scratch,
    )
    def sc_kernel(x_hbm, idx_hbm, tab_hbm, comp_hbm, res_hbm, *scr):
        idx_all = scr[0]
        x_bufs = scr[1:1 + RING]
        r_bufs = scr[1 + RING:1 + 2 * RING]
        sems = scr[1 + 2 * RING:]
        in_x = sems[0:RING]
        in_g = sems[RING:2 * RING]
        out_c = sems[2 * RING:3 * RING]
        out_r = sems[3 * RING:]

        wid = lax.axis_index("s") * NC + lax.axis_index("c")
        base = wid * rows_per_w
        pltpu.sync_copy(idx_hbm.at[pl.ds(base, rows_per_w)], idx_all)

        def issue_inputs(g, b):
            r0 = base + g * CHUNK
            pltpu.async_copy(
                tab_hbm.at[idx_all.at[pl.ds(g * CHUNK, CHUNK)]],
                r_bufs[b], in_g[b])
            pltpu.async_copy(x_hbm.at[pl.ds(r0, CHUNK)], x_bufs[b], in_x[b])

        def wait_inputs(b):
            pltpu.make_async_copy(
                x_hbm.at[pl.ds(0, CHUNK)], x_bufs[b], in_x[b]).wait()
            pltpu.make_async_copy(
                tab_hbm.at[idx_all.at[pl.ds(0, CHUNK)]],
                r_bufs[b], in_g[b]).wait()

        def issue_outputs(g, b):
            r0 = base + g * CHUNK
            pltpu.async_copy(r_bufs[b], comp_hbm.at[pl.ds(r0, CHUNK)], out_c[b])
            pltpu.async_copy(x_bufs[b], res_hbm.at[pl.ds(r0, CHUNK)], out_r[b])

        def wait_outputs(b):
            pltpu.make_async_copy(
                r_bufs[b], comp_hbm.at[pl.ds(0, CHUNK)], out_c[b]).wait()
            pltpu.make_async_copy(
                x_bufs[b], res_hbm.at[pl.ds(0, CHUNK)], out_r[b]).wait()

        def compute(b):
            def row(r, c):
                def col(j, c2):
                    sl = pl.ds(j * L, L)
                    x_bufs[b][r, sl] = x_bufs[b][r, sl] - r_bufs[b][r, sl]
                    return c2

                return lax.fori_loop(0, D // L, col, c, unroll=48)

            lax.fori_loop(0, CHUNK, row, 0)

        for p in range(PF):
            issue_inputs(p, p)

        def giter(G, carry):
            for b in range(RING):
                g = G * RING + b
                gp = g + PF
                pb = (b + PF) % RING
                wait_inputs(b)
                @pl.when(gp < steps)
                def _():
                    @pl.when(gp >= RING)
                    def _():
                        wait_outputs(pb)

                    issue_inputs(gp, pb)

                compute(b)
                issue_outputs(g, b)
            return carry

        lax.fori_loop(0, g_iters, giter, 0)
        for b in range(RING):
            wait_outputs(b)

    return sc_kernel


def kernel(x, cycle_indices, learnable_cycles):
    B, S, d = x.shape
    N = B * S
    x2 = x.reshape(N, d)
    idx = cycle_indices.reshape(N).astype(jnp.int32)
    comp, res = _make_sc_kernel(N)(x2, idx, learnable_cycles)
    return comp.reshape(B, S, d), res.reshape(B, S, d)
